# Initial kernel scaffold; baseline (speedup 1.0000x reference)
import jax
import jax.numpy as jnp
from jax.experimental import pallas as pl


def kernel(hidden_states, W_gate, ln_scale, ln_bias, W_up, conv_w, conv_b, W_q, W_k, W_v, w_i, b_i, w_f, b_f, skip, mh_scale, W_down):
    raise NotImplementedError("write your pallas kernel here")


# speedup vs baseline: 3.1045x; 3.1045x over previous
import functools
import math

import jax
import jax.numpy as jnp
from jax.experimental import pallas as pl
from jax.experimental.pallas import tpu as pltpu

B, S, D = 1, 2048, 768
E, TOP_K = 8, 2
DI = 2 * D
H = 4
DH = DI // H
K_CONV = 4
T = B * S
KT = TOP_K * T

TILE = 256
NT = KT // TILE + E
P = NT * TILE


def _router_body(x_ref, wg_ref, logits_ref, topi_ref, topw_ref):
    x = x_ref[...]
    logits = jnp.dot(x, wg_ref[...], preferred_element_type=jnp.float32)
    logits_ref[...] = logits
    idx = jax.lax.broadcasted_iota(jnp.int32, (T, E), 1)
    m1 = jnp.max(logits, axis=1, keepdims=True)
    a1 = jnp.min(jnp.where(logits == m1, idx, E), axis=1, keepdims=True)
    masked = jnp.where(idx == a1, -jnp.inf, logits)
    m2 = jnp.max(masked, axis=1, keepdims=True)
    a2 = jnp.min(jnp.where((logits == m2) & (idx != a1), idx, E), axis=1,
                 keepdims=True)
    topi_ref[...] = jnp.concatenate([a1, a2], axis=1)
    w0 = jax.nn.sigmoid(m1 - m2)
    topw_ref[...] = jnp.concatenate([w0, 1.0 - w0], axis=1)


def _router_call(x, W_gate):
    return pl.pallas_call(
        _router_body,
        out_shape=(
            jax.ShapeDtypeStruct((T, E), jnp.float32),
            jax.ShapeDtypeStruct((T, TOP_K), jnp.int32),
            jax.ShapeDtypeStruct((T, TOP_K), jnp.float32),
        ),
    )(x, W_gate)


def _up_body(te_ref, rt_ref, x_ref, lns_ref, lnb_ref, wup_ref, cw_ref, cb_ref,
             xm_ref, xc_ref, sz_ref, xg_ref):
    j = pl.program_id(0)

    def gather_one(i, carry):
        t = rt_ref[j * TILE + i]
        xg_ref[pl.ds(i, 1), :] = x_ref[pl.ds(t, 1), :]
        return carry

    jax.lax.fori_loop(0, TILE, gather_one, 0)

    xg = xg_ref[...]
    mu = jnp.mean(xg, axis=1, keepdims=True)
    var = jnp.mean((xg - mu) * (xg - mu), axis=1, keepdims=True)
    xn = (xg - mu) / jnp.sqrt(var + 1e-5)
    xn = xn * lns_ref[0, 0] + lnb_ref[0, 0]
    up = jnp.dot(xn, wup_ref[0], preferred_element_type=jnp.float32)
    x_m = up[:, :DI]
    z = up[:, DI:]
    xc = jax.nn.silu(x_m * cw_ref[0, 0] + cb_ref[0, 0])
    xm_ref[...] = x_m
    xc_ref[...] = xc
    sz_ref[...] = z * jax.nn.sigmoid(z)


def _up_call(x, tile_expert, row_token, ln_scale, ln_bias, W_up, conv_w,
             conv_b):
    grid_spec = pltpu.PrefetchScalarGridSpec(
        num_scalar_prefetch=2,
        grid=(NT,),
        in_specs=[
            pl.BlockSpec((T, D), lambda j, te, rt: (0, 0)),
            pl.BlockSpec((1, 1, D), lambda j, te, rt: (te[j], 0, 0)),
            pl.BlockSpec((1, 1, D), lambda j, te, rt: (te[j], 0, 0)),
            pl.BlockSpec((1, D, 2 * DI), lambda j, te, rt: (te[j], 0, 0)),
            pl.BlockSpec((1, 1, DI), lambda j, te, rt: (te[j], 0, 0)),
            pl.BlockSpec((1, 1, DI), lambda j, te, rt: (te[j], 0, 0)),
        ],
        out_specs=[
            pl.BlockSpec((TILE, DI), lambda j, te, rt: (j, 0)),
            pl.BlockSpec((TILE, DI), lambda j, te, rt: (j, 0)),
            pl.BlockSpec((TILE, DI), lambda j, te, rt: (j, 0)),
        ],
        scratch_shapes=[pltpu.VMEM((TILE, D), jnp.float32)],
    )
    return pl.pallas_call(
        _up_body,
        grid_spec=grid_spec,
        out_shape=(
            jax.ShapeDtypeStruct((P, DI), jnp.float32),
            jax.ShapeDtypeStruct((P, DI), jnp.float32),
            jax.ShapeDtypeStruct((P, DI), jnp.float32),
        ),
        compiler_params=pltpu.CompilerParams(
            vmem_limit_bytes=100 * 1024 * 1024),
    )(tile_expert, row_token, x, ln_scale.reshape(E, 1, D),
      ln_bias.reshape(E, 1, D), W_up,
      conv_w[:, K_CONV - 1, :].reshape(E, 1, DI), conv_b.reshape(E, 1, DI))


def _proj_body(te_ref, in_ref, w_ref, o_ref):
    o_ref[...] = jnp.dot(in_ref[...], w_ref[0],
                         preferred_element_type=jnp.float32)


def _proj_call(inp, W, tile_expert):
    n = W.shape[-1]
    grid_spec = pltpu.PrefetchScalarGridSpec(
        num_scalar_prefetch=1,
        grid=(NT,),
        in_specs=[
            pl.BlockSpec((TILE, DI), lambda j, te: (j, 0)),
            pl.BlockSpec((1, DI, n), lambda j, te: (te[j], 0, 0)),
        ],
        out_specs=pl.BlockSpec((TILE, n), lambda j, te: (j, 0)),
    )
    return pl.pallas_call(
        _proj_body,
        grid_spec=grid_spec,
        out_shape=jax.ShapeDtypeStruct((P, n), jnp.float32),
    )(tile_expert, inp, W)


def _cell_body(te_ref, q_ref, k_ref, v_ref, xc_ref, sz_ref, wi_ref,
               bi_ref, skip_ref, mhs_ref, wd_ref, y_ref):
    xc = xc_ref[...]
    q = q_ref[...]
    k = k_ref[...]
    v = v_ref[...]
    wi = wi_ref[0]
    ipre = (jnp.dot(q, wi[:DI], preferred_element_type=jnp.float32)
            + jnp.dot(k, wi[DI:2 * DI], preferred_element_type=jnp.float32)
            + jnp.dot(v, wi[2 * DI:], preferred_element_type=jnp.float32)
            + bi_ref[0, 0])
    inv_sqrt_dh = 1.0 / math.sqrt(DH)
    heads = []
    for h in range(H):
        qh = q[:, h * DH:(h + 1) * DH]
        kh = k[:, h * DH:(h + 1) * DH]
        vh = v[:, h * DH:(h + 1) * DH]
        qk = jnp.sum(qh * kh, axis=1, keepdims=True) * inv_sqrt_dh
        ih = ipre[:, h:h + 1]
        n = jnp.maximum(jnp.abs(qk), jnp.exp(-ih))
        hv = (qk / n) * vh
        hmu = jnp.mean(hv, axis=1, keepdims=True)
        hvar = jnp.mean((hv - hmu) * (hv - hmu), axis=1, keepdims=True)
        hn = (hv - hmu) / jnp.sqrt(hvar + 1e-5)
        heads.append(hn * mhs_ref[0, 0, h * DH:(h + 1) * DH])
    hn_all = jnp.concatenate(heads, axis=1)
    hs = hn_all + skip_ref[0, 0] * xc
    ho = hs * sz_ref[...]
    y_ref[...] = jnp.dot(ho, wd_ref[0], preferred_element_type=jnp.float32)


def _cell_call(q, k, v, xc, sz, tile_expert, w_i, b_i, skip, mh_scale,
               W_down):
    grid_spec = pltpu.PrefetchScalarGridSpec(
        num_scalar_prefetch=1,
        grid=(NT,),
        in_specs=[
            pl.BlockSpec((TILE, DI), lambda j, te: (j, 0)),
            pl.BlockSpec((TILE, DI), lambda j, te: (j, 0)),
            pl.BlockSpec((TILE, DI), lambda j, te: (j, 0)),
            pl.BlockSpec((TILE, DI), lambda j, te: (j, 0)),
            pl.BlockSpec((TILE, DI), lambda j, te: (j, 0)),
            pl.BlockSpec((1, 3 * DI, H), lambda j, te: (te[j], 0, 0)),
            pl.BlockSpec((1, 1, H), lambda j, te: (te[j], 0, 0)),
            pl.BlockSpec((1, 1, DI), lambda j, te: (te[j], 0, 0)),
            pl.BlockSpec((1, 1, DI), lambda j, te: (te[j], 0, 0)),
            pl.BlockSpec((1, DI, D), lambda j, te: (te[j], 0, 0)),
        ],
        out_specs=pl.BlockSpec((TILE, D), lambda j, te: (j, 0)),
    )
    return pl.pallas_call(
        _cell_body,
        grid_spec=grid_spec,
        out_shape=jax.ShapeDtypeStruct((P, D), jnp.float32),
    )(tile_expert, q, k, v, xc, sz, w_i, b_i.reshape(E, 1, H),
      skip.reshape(E, 1, DI), mh_scale.reshape(E, 1, DI), W_down)


def _combine_body(pos_ref, w_ref, x_ref, y_ref, o_ref):
    j = pl.program_id(0)

    def one(i, carry):
        t = j * TILE + i
        p0 = pos_ref[2 * t]
        p1 = pos_ref[2 * t + 1]
        w0 = w_ref[2 * t]
        w1 = w_ref[2 * t + 1]
        o_ref[pl.ds(i, 1), :] = (x_ref[pl.ds(i, 1), :]
                                 + w0 * y_ref[pl.ds(p0, 1), :]
                                 + w1 * y_ref[pl.ds(p1, 1), :])
        return carry

    jax.lax.fori_loop(0, TILE, one, 0)


def _combine_call(x, y, pos, topw_flat):
    grid_spec = pltpu.PrefetchScalarGridSpec(
        num_scalar_prefetch=2,
        grid=(T // TILE,),
        in_specs=[
            pl.BlockSpec((TILE, D), lambda j, p, w: (j, 0)),
            pl.BlockSpec((P, D), lambda j, p, w: (0, 0)),
        ],
        out_specs=pl.BlockSpec((TILE, D), lambda j, p, w: (j, 0)),
    )
    return pl.pallas_call(
        _combine_body,
        grid_spec=grid_spec,
        out_shape=jax.ShapeDtypeStruct((T, D), jnp.float32),
        compiler_params=pltpu.CompilerParams(
            vmem_limit_bytes=100 * 1024 * 1024),
    )(pos, topw_flat, x, y)


@jax.jit
def kernel(hidden_states, W_gate, ln_scale, ln_bias, W_up, conv_w, conv_b,
           W_q, W_k, W_v, w_i, b_i, w_f, b_f, skip, mh_scale, W_down):
    x = hidden_states.reshape(T, D)
    logits, topi, topw = _router_call(x, W_gate)

    flat_e = topi.reshape(-1)
    perm = jnp.argsort(flat_e, stable=True)
    sorted_e = flat_e[perm]
    counts = jnp.bincount(flat_e, length=E).astype(jnp.int32)
    tiles_pe = (counts + TILE - 1) // TILE
    cum_tiles = jnp.cumsum(tiles_pe)
    tiles_before = cum_tiles - tiles_pe
    offs = tiles_before * TILE
    cstart = jnp.cumsum(counts) - counts
    rank = jnp.arange(KT, dtype=jnp.int32) - cstart[sorted_e]
    dest = offs[sorted_e] + rank
    row_token = jnp.zeros((P,), jnp.int32).at[dest].set(
        (perm // TOP_K).astype(jnp.int32))
    pos = jnp.zeros((KT,), jnp.int32).at[perm].set(dest.astype(jnp.int32))
    tile_expert = jnp.minimum(
        jnp.searchsorted(cum_tiles, jnp.arange(NT, dtype=jnp.int32),
                         side='right'),
        E - 1).astype(jnp.int32)

    xm, xc, sz = _up_call(x, tile_expert, row_token, ln_scale, ln_bias, W_up,
                          conv_w, conv_b)
    q = _proj_call(xc, W_q, tile_expert)
    k = _proj_call(xc, W_k, tile_expert)
    v = _proj_call(xm, W_v, tile_expert)
    y = _cell_call(q, k, v, xc, sz, tile_expert, w_i, b_i, skip,
                   mh_scale, W_down)
    out = _combine_call(x, y, pos, topw.reshape(-1))
    return out.reshape(B, S, D), logits
